# baseline (speedup 1.0000x reference)
import functools
import math

import jax
import jax.numpy as jnp
import numpy as np
from jax import lax
from jax.experimental import pallas as pl
from jax.experimental.pallas import tpu as pltpu
from jax.experimental.pallas import tpu_sc as plsc

VOCAB = 1000000
D_MODEL = 64
B = 4096
L = 200
N = B * L

NUM_CORES = 2
NUM_SUBCORES = 16
NUM_WORKERS = NUM_CORES * NUM_SUBCORES
ROWS_PER_WORKER = N // NUM_WORKERS
SEQS_PER_WORKER = ROWS_PER_WORKER // L
LANES = 16
SPLIT_A = 104
SPLIT_B = L - SPLIT_A


def _make_pe(max_len, d_model):
    pos = np.arange(max_len, dtype=np.float32)[:, None]
    div = np.exp(
        np.arange(0, d_model, 2, dtype=np.float32) * (-math.log(10000.0) / d_model)
    )
    pe = np.zeros((max_len, d_model), dtype=np.float32)
    pe[:, 0::2] = np.sin(pos * div)
    pe[:, 1::2] = np.cos(pos * div)
    return pe


_PE = _make_pe(L, D_MODEL)


@functools.partial(
    pl.kernel,
    mesh=plsc.VectorSubcoreMesh(core_axis_name="c", subcore_axis_name="s"),
    compiler_params=pltpu.CompilerParams(use_tc_tiling_on_sc=False),
    out_type=jax.ShapeDtypeStruct((N, D_MODEL), jnp.float32),
    scratch_types=[
        pltpu.VMEM((SPLIT_A,), jnp.int32),
        pltpu.VMEM((SPLIT_B,), jnp.int32),
        pltpu.VMEM((L, D_MODEL), jnp.float32),
        pltpu.VMEM((L, D_MODEL), jnp.float32),
        pltpu.SemaphoreType.DMA,
        pltpu.SemaphoreType.DMA,
    ],
)
def _emb_lookup(ids_hbm, table_hbm, pe_hbm, out_hbm,
                idx_a, idx_b, rows_v, pe_v, sem_a, sem_b):
    wid = lax.axis_index("s") * NUM_CORES + lax.axis_index("c")
    base = wid * ROWS_PER_WORKER

    pltpu.sync_copy(pe_hbm, pe_v)

    def seq_body(g, _):
        off = base + g * L
        pltpu.sync_copy(ids_hbm.at[pl.ds(off, SPLIT_A)], idx_a)
        pltpu.sync_copy(ids_hbm.at[pl.ds(off + SPLIT_A, SPLIT_B)], idx_b)
        cp_a = pltpu.async_copy(
            table_hbm.at[idx_a], rows_v.at[pl.ds(0, SPLIT_A)], sem_a)
        cp_b = pltpu.async_copy(
            table_hbm.at[idx_b], rows_v.at[pl.ds(SPLIT_A, SPLIT_B)], sem_b)
        cp_a.wait()
        cp_b.wait()

        def row_body(p, _):
            for j in range(D_MODEL // LANES):
                sl = pl.ds(j * LANES, LANES)
                rows_v[p, sl] = rows_v[p, sl] * 8.0 + pe_v[p, sl]
            return _

        lax.fori_loop(0, L, row_body, None, unroll=False)
        pltpu.sync_copy(rows_v, out_hbm.at[pl.ds(off, L)])
        return _

    lax.fori_loop(0, SEQS_PER_WORKER, seq_body, None, unroll=False)


def kernel(input_ids, W):
    ids_flat = input_ids.reshape(-1).astype(jnp.int32)
    out = _emb_lookup(ids_flat, W, jnp.asarray(_PE))
    return out.reshape(B, L, D_MODEL)


# speedup vs baseline: 1.2621x; 1.2621x over previous
import functools
import math

import jax
import jax.numpy as jnp
import numpy as np
from jax import lax
from jax.experimental import pallas as pl
from jax.experimental.pallas import tpu as pltpu
from jax.experimental.pallas import tpu_sc as plsc

VOCAB = 1000000
D_MODEL = 64
B = 4096
L = 200
N = B * L

NUM_CORES = 2
NUM_SUBCORES = 16
NUM_WORKERS = NUM_CORES * NUM_SUBCORES
ROWS_PER_WORKER = N // NUM_WORKERS
NCHUNK = ROWS_PER_WORKER // L
LANES = 16
SPLIT_A = 104
SPLIT_B = L - SPLIT_A
NGBUF = 4
NOBUF = 2
CHUNK_ELTS = L * D_MODEL


def _make_pe(max_len, d_model):
    pos = np.arange(max_len, dtype=np.float32)[:, None]
    div = np.exp(
        np.arange(0, d_model, 2, dtype=np.float32) * (-math.log(10000.0) / d_model)
    )
    pe = np.zeros((max_len, d_model), dtype=np.float32)
    pe[:, 0::2] = np.sin(pos * div)
    pe[:, 1::2] = np.cos(pos * div)
    return pe


_PE = _make_pe(L, D_MODEL)


@functools.partial(
    pl.kernel,
    mesh=plsc.VectorSubcoreMesh(core_axis_name="c", subcore_axis_name="s"),
    compiler_params=pltpu.CompilerParams(use_tc_tiling_on_sc=False),
    out_type=jax.ShapeDtypeStruct((N * D_MODEL,), jnp.float32),
    scratch_types=[
        pltpu.VMEM((ROWS_PER_WORKER,), jnp.int32),
        pltpu.VMEM((NGBUF, L, D_MODEL), jnp.float32),
        pltpu.VMEM((NOBUF, CHUNK_ELTS), jnp.float32),
        pltpu.VMEM((CHUNK_ELTS,), jnp.float32),
        [pltpu.SemaphoreType.DMA] * NGBUF,
        [pltpu.SemaphoreType.DMA] * NGBUF,
        [pltpu.SemaphoreType.DMA] * NOBUF,
    ],
)
def _emb_lookup(ids_hbm, table_hbm, pe_hbm, out_hbm,
                idx_v, rows_v, stage_v, pe_v, sem_ga, sem_gb, sem_out):
    wid = lax.axis_index("s") * NUM_CORES + lax.axis_index("c")
    base = wid * ROWS_PER_WORKER

    pltpu.sync_copy(ids_hbm.at[pl.ds(base, ROWS_PER_WORKER)], idx_v)
    pltpu.sync_copy(pe_hbm, pe_v)

    def issue_gather(g, b):
        loc = g * L
        pltpu.async_copy(
            table_hbm.at[idx_v.at[pl.ds(loc, SPLIT_A)]],
            rows_v.at[b, pl.ds(0, SPLIT_A)], sem_ga[b])
        pltpu.async_copy(
            table_hbm.at[idx_v.at[pl.ds(loc + SPLIT_A, SPLIT_B)]],
            rows_v.at[b, pl.ds(SPLIT_A, SPLIT_B)], sem_gb[b])

    def wait_gather(b):
        pltpu.make_async_copy(
            table_hbm.at[pl.ds(0, SPLIT_A)],
            rows_v.at[b, pl.ds(0, SPLIT_A)], sem_ga[b]).wait()
        pltpu.make_async_copy(
            table_hbm.at[pl.ds(0, SPLIT_B)],
            rows_v.at[b, pl.ds(SPLIT_A, SPLIT_B)], sem_gb[b]).wait()

    def wait_out(bo):
        pltpu.make_async_copy(
            stage_v.at[bo], out_hbm.at[pl.ds(0, CHUNK_ELTS)],
            sem_out[bo]).wait()

    def process(g, b, bo, drain_out):
        wait_gather(b)
        if drain_out:
            wait_out(bo)

        def row_body(p, _):
            o = p * D_MODEL
            for j in range(D_MODEL // LANES):
                c = j * LANES
                stage_v[bo, pl.ds(o + c, LANES)] = (
                    rows_v[b, p, pl.ds(c, LANES)] * 8.0
                    + pe_v[pl.ds(o + c, LANES)])
            return _

        lax.fori_loop(0, L, row_body, None, unroll=False)
        pltpu.async_copy(
            stage_v.at[bo],
            out_hbm.at[pl.ds((base + g * L) * D_MODEL, CHUNK_ELTS)],
            sem_out[bo])

    for b in range(NGBUF - 1):
        issue_gather(b, b)
    issue_gather(3, 3)
    process(0, 0, 0, drain_out=False)
    issue_gather(4, 0)
    process(1, 1, 1, drain_out=False)
    issue_gather(5, 1)
    process(2, 2, 0, drain_out=True)
    issue_gather(6, 2)
    process(3, 3, 1, drain_out=True)

    def block_body(tt, _):
        g0 = tt * NGBUF
        for b in range(NGBUF):
            issue_gather(g0 + b + NGBUF - 1, (b + NGBUF - 1) % NGBUF)
            process(g0 + b, b, b % NOBUF, drain_out=True)
        return _

    lax.fori_loop(1, NCHUNK // NGBUF - 1, block_body, None, unroll=False)

    g0 = NCHUNK - NGBUF
    issue_gather(NCHUNK - 1, NGBUF - 1)
    for b in range(NGBUF):
        process(g0 + b, b, b % NOBUF, drain_out=True)
    for bo in range(NOBUF):
        wait_out(bo)


def kernel(input_ids, W):
    ids_flat = input_ids.reshape(-1).astype(jnp.int32)
    out = _emb_lookup(ids_flat, W, jnp.asarray(_PE).reshape(-1))
    return out.reshape(B, L, D_MODEL)
